# Initial kernel scaffold; baseline (speedup 1.0000x reference)
import jax
import jax.numpy as jnp
from jax.experimental import pallas as pl


def kernel(x, mask, ln1_g, ln1_b, wq, bq, wk, bk, wv, bv, wo, bo, ln2_g, ln2_b, w1, b1, w2, b2):
    raise NotImplementedError("write your pallas kernel here")


# speedup vs baseline: 1.5110x; 1.5110x over previous
import functools
import math

import jax
import jax.numpy as jnp
from jax.experimental import pallas as pl
from jax.experimental.pallas import tpu as pltpu

_NEG_INF = -1e30
_LN_EPS = 1e-6


def _layer_norm_f32(z, gamma, beta):
    mu = jnp.mean(z, axis=-1, keepdims=True)
    var = jnp.mean(jnp.square(z - mu), axis=-1, keepdims=True)
    return (z - mu) * jax.lax.rsqrt(var + _LN_EPS) * gamma + beta


def _encoder_layer_kernel(
    x_ref, kbias_ref,
    ln1_g_ref, ln1_b_ref,
    wqkv_ref, bqkv_ref, wo_ref, bo_ref,
    ln2_g_ref, ln2_b_ref,
    w1_ref, b1_ref, w2_ref, b2_ref,
    o_ref,
    *, num_heads: int, head_size: int, block_batch: int, seq_len: int,
):
    x = x_ref[...].astype(jnp.float32)
    D = num_heads * head_size
    bb, T = block_batch, seq_len

    xn = _layer_norm_f32(x, ln1_g_ref[...], ln1_b_ref[...])
    xn_b = xn.astype(jnp.bfloat16)

    qkv = jnp.dot(xn_b, wqkv_ref[...],
                  preferred_element_type=jnp.float32) + bqkv_ref[...]

    kbias = kbias_ref[...]

    ctxs = []
    for h in range(num_heads):
        lo = h * head_size
        qh = qkv[:, lo:lo + head_size].astype(jnp.bfloat16).reshape(bb, T, head_size)
        kh = qkv[:, D + lo:D + lo + head_size].astype(jnp.bfloat16).reshape(bb, T, head_size)
        vh = qkv[:, 2 * D + lo:2 * D + lo + head_size].astype(jnp.bfloat16).reshape(bb, T, head_size)
        s = jax.lax.dot_general(qh, kh, (((2,), (2,)), ((0,), (0,))),
                                preferred_element_type=jnp.float32)
        s = s + kbias
        s = s - jnp.max(s, axis=-1, keepdims=True)
        p = jnp.exp(s)
        p = p * pl.reciprocal(jnp.sum(p, axis=-1, keepdims=True), approx=True)
        ctx = jax.lax.dot_general(p.astype(jnp.bfloat16), vh,
                                  (((2,), (1,)), ((0,), (0,))),
                                  preferred_element_type=jnp.float32)
        ctxs.append(ctx.reshape(bb * T, head_size).astype(jnp.bfloat16))
    ctx_full = jnp.concatenate(ctxs, axis=-1)

    h1 = x + bo_ref[...] + jnp.dot(ctx_full, wo_ref[...],
                                   preferred_element_type=jnp.float32)

    hn = _layer_norm_f32(h1, ln2_g_ref[...], ln2_b_ref[...])
    f = jnp.dot(hn.astype(jnp.bfloat16), w1_ref[...],
                preferred_element_type=jnp.float32) + b1_ref[...]
    f = jnp.maximum(f, 0.0)
    f = jnp.dot(f.astype(jnp.bfloat16), w2_ref[...],
                preferred_element_type=jnp.float32) + b2_ref[...]

    o_ref[...] = (f + h1).astype(o_ref.dtype)


def kernel(x, mask, ln1_g, ln1_b, wq, bq, wk, bk, wv, bv, wo, bo,
           ln2_g, ln2_b, w1, b1, w2, b2):
    num_heads = 12
    B, T, D = x.shape
    head_size = D // num_heads
    F = w1.shape[1]

    block_batch = max(1, min(B, 512 // max(T, 1)))
    while block_batch > 1 and (B % block_batch != 0 or (block_batch * T) % 8 != 0):
        block_batch -= 1
    M = block_batch * T
    num_blocks = B // block_batch

    scale = 1.0 / math.sqrt(head_size)
    wqkv = jnp.concatenate([wq * scale, wk, wv], axis=1).astype(jnp.bfloat16)
    bqkv = jnp.concatenate([bq * scale, bk, bv], axis=1)
    wo_b = wo.astype(jnp.bfloat16)
    w1_b = w1.astype(jnp.bfloat16)
    w2_b = w2.astype(jnp.bfloat16)

    mask_full = jnp.concatenate([mask[:, :, 0:1], mask], axis=-1)
    kbias = jnp.where(mask_full, 0.0, _NEG_INF).astype(jnp.float32)

    x2 = x.reshape(B * T, D)
    const = lambda b: (0, 0)
    _kernel_fn = functools.partial(
        _encoder_layer_kernel, num_heads=num_heads, head_size=head_size,
        block_batch=block_batch, seq_len=T)

    out = pl.pallas_call(
        _kernel_fn,
        out_shape=jax.ShapeDtypeStruct((B * T, D), x.dtype),
        grid=(num_blocks,),
        in_specs=[
            pl.BlockSpec((M, D), lambda b: (b, 0)),
            pl.BlockSpec((block_batch, 1, T), lambda b: (b, 0, 0)),
            pl.BlockSpec((1, D), const),
            pl.BlockSpec((1, D), const),
            pl.BlockSpec((D, 3 * D), const),
            pl.BlockSpec((1, 3 * D), const),
            pl.BlockSpec((D, D), const),
            pl.BlockSpec((1, D), const),
            pl.BlockSpec((1, D), const),
            pl.BlockSpec((1, D), const),
            pl.BlockSpec((D, F), const),
            pl.BlockSpec((1, F), const),
            pl.BlockSpec((F, D), const),
            pl.BlockSpec((1, D), const),
        ],
        out_specs=pl.BlockSpec((M, D), lambda b: (b, 0)),
        compiler_params=pltpu.CompilerParams(
            dimension_semantics=("parallel",),
            vmem_limit_bytes=60 * 1024 * 1024),
    )(x2, kbias,
      ln1_g, ln1_b,
      wqkv, bqkv, wo_b, bo,
      ln2_g, ln2_b,
      w1_b, b1, w2_b, b2)
    return out.reshape(B, T, D)


# speedup vs baseline: 1.9006x; 1.2578x over previous
import functools
import math

import jax
import jax.numpy as jnp
from jax.experimental import pallas as pl
from jax.experimental.pallas import tpu as pltpu

_NEG_INF = -1e30
_LN_EPS = 1e-6


def _layer_norm_f32(z, gamma, beta):
    mu = jnp.mean(z, axis=-1, keepdims=True)
    var = jnp.mean(jnp.square(z - mu), axis=-1, keepdims=True)
    return (z - mu) * jax.lax.rsqrt(var + _LN_EPS) * gamma + beta


def _encoder_layer_kernel(
    x_ref, kbias_ref,
    ln1_g_ref, ln1_b_ref,
    wqkv_ref, bqkv_ref, wo_ref, bo_ref,
    ln2_g_ref, ln2_b_ref,
    w1_ref, b1_ref, w2_ref, b2_ref,
    o_ref,
    h1_scr,
    *, num_heads: int, head_size: int, block_batch: int, seq_len: int,
    ffn_chunks: int,
):
    D = num_heads * head_size
    bb, T = block_batch, seq_len
    M = bb * T
    Mc = M // ffn_chunks


    def _ffn_chunk(c):
        rows = pl.ds(c * Mc, Mc)
        h1c = h1_scr[rows, :]
        hnc = _layer_norm_f32(h1c, ln2_g_ref[...], ln2_b_ref[...])
        fc = jnp.dot(hnc.astype(jnp.bfloat16), w1_ref[...],
                     preferred_element_type=jnp.float32) + b1_ref[...]
        fc = jnp.maximum(fc, 0.0)
        oc = jnp.dot(fc.astype(jnp.bfloat16), w2_ref[...],
                     preferred_element_type=jnp.float32) + b2_ref[...]
        o_ref[rows, :] = (oc + h1c).astype(o_ref.dtype)

    _ffn_chunk(0)

    x = x_ref[...].astype(jnp.float32)
    xn = _layer_norm_f32(x, ln1_g_ref[...], ln1_b_ref[...])
    qkv = jnp.dot(xn.astype(jnp.bfloat16), wqkv_ref[...],
                  preferred_element_type=jnp.float32) + bqkv_ref[...]

    kbias = kbias_ref[...]
    _chunk_at = {1: 1, 2: 2, 4: 3, 5: 4, 7: 5, 8: 6, 10: 7} if ffn_chunks == 8 \
        else {3: 1, 6: 2, 9: 3}
    ctxs = []
    for h in range(num_heads):
        if h in _chunk_at:
            _ffn_chunk(_chunk_at[h])
        lo = h * head_size
        qh = qkv[:, lo:lo + head_size].astype(jnp.bfloat16).reshape(bb, T, head_size)
        kh = qkv[:, D + lo:D + lo + head_size].astype(jnp.bfloat16).reshape(bb, T, head_size)
        vh = qkv[:, 2 * D + lo:2 * D + lo + head_size].astype(jnp.bfloat16).reshape(bb, T, head_size)
        s = jax.lax.dot_general(qh, kh, (((2,), (2,)), ((0,), (0,))),
                                preferred_element_type=jnp.float32)
        s = s + kbias
        p = jnp.exp(jnp.minimum(s, 80.0))
        r = pl.reciprocal(jnp.sum(p, axis=-1, keepdims=True), approx=True)
        ctx = jax.lax.dot_general(p.astype(jnp.bfloat16), vh,
                                  (((2,), (1,)), ((0,), (0,))),
                                  preferred_element_type=jnp.float32)
        ctx = ctx * r
        ctxs.append(ctx.reshape(M, head_size).astype(jnp.bfloat16))
    ctx_full = jnp.concatenate(ctxs, axis=-1)

    h1_scr[...] = x + bo_ref[...] + jnp.dot(ctx_full, wo_ref[...],
                                            preferred_element_type=jnp.float32)


def kernel(x, mask, ln1_g, ln1_b, wq, bq, wk, bk, wv, bv, wo, bo,
           ln2_g, ln2_b, w1, b1, w2, b2):
    num_heads = 12
    B, T, D = x.shape
    head_size = D // num_heads
    F = w1.shape[1]

    block_batch = max(1, min(B, 512 // max(T, 1)))
    while block_batch > 1 and (B % block_batch != 0 or (block_batch * T) % 8 != 0):
        block_batch -= 1
    M = block_batch * T
    num_blocks = B // block_batch
    num_cores = 2 if num_blocks % 2 == 0 else 1
    steps_per_core = num_blocks // num_cores
    ffn_chunks = 4 if M % (4 * 8) == 0 else 1

    scale = 1.0 / math.sqrt(head_size)
    wqkv = jnp.concatenate([wq * scale, wk, wv], axis=1).astype(jnp.bfloat16)
    bqkv = jnp.concatenate([bq * scale, bk, bv], axis=1)
    wo_b = wo.astype(jnp.bfloat16)
    w1_b = w1.astype(jnp.bfloat16)
    w2_b = w2.astype(jnp.bfloat16)

    mask_full = jnp.concatenate([mask[:, :, 0:1], mask], axis=-1)
    kbias = jnp.where(mask_full, 0.0, _NEG_INF).astype(jnp.float32)

    x2 = x.reshape(B * T, D)
    spc = steps_per_core
    cur = lambda c, j: (c * spc + jnp.minimum(j, spc - 1), 0)
    prev = lambda c, j: (c * spc + jnp.maximum(j - 1, 0), 0)
    cur3 = lambda c, j: (c * spc + jnp.minimum(j, spc - 1), 0, 0)
    const = lambda c, j: (0, 0)

    _kernel_fn = functools.partial(
        _encoder_layer_kernel, num_heads=num_heads, head_size=head_size,
        block_batch=block_batch, seq_len=T, ffn_chunks=ffn_chunks)

    out = pl.pallas_call(
        _kernel_fn,
        out_shape=jax.ShapeDtypeStruct((B * T, D), x.dtype),
        grid=(num_cores, spc + 1),
        in_specs=[
            pl.BlockSpec((M, D), cur),
            pl.BlockSpec((block_batch, 1, T), cur3),
            pl.BlockSpec((1, D), const),
            pl.BlockSpec((1, D), const),
            pl.BlockSpec((D, 3 * D), const),
            pl.BlockSpec((1, 3 * D), const),
            pl.BlockSpec((D, D), const),
            pl.BlockSpec((1, D), const),
            pl.BlockSpec((1, D), const),
            pl.BlockSpec((1, D), const),
            pl.BlockSpec((D, F), const),
            pl.BlockSpec((1, F), const),
            pl.BlockSpec((F, D), const),
            pl.BlockSpec((1, D), const),
        ],
        out_specs=pl.BlockSpec((M, D), prev),
        scratch_shapes=[pltpu.VMEM((M, D), jnp.float32)],
        compiler_params=pltpu.CompilerParams(
            dimension_semantics=("parallel", "arbitrary"),
            vmem_limit_bytes=60 * 1024 * 1024),
    )(x2, kbias,
      ln1_g, ln1_b,
      wqkv, bqkv, wo_b, bo,
      ln2_g, ln2_b,
      w1_b, b1, w2_b, b2)
    return out.reshape(B, T, D)


# speedup vs baseline: 2.0195x; 1.0626x over previous
import functools
import math

import jax
import jax.numpy as jnp
from jax.experimental import pallas as pl
from jax.experimental.pallas import tpu as pltpu

_NEG_INF = -1e30
_LN_EPS = 1e-6


def _layer_norm_f32(z, gamma, beta):
    mu = jnp.mean(z, axis=-1, keepdims=True)
    var = jnp.mean(jnp.square(z - mu), axis=-1, keepdims=True)
    return (z - mu) * jax.lax.rsqrt(var + _LN_EPS) * gamma + beta


def _encoder_layer_kernel(
    x_ref, kbias_ref,
    ln1_g_ref, ln1_b_ref,
    wqkv_ref, bqkv_ref, wo_ref, bo_ref,
    ln2_g_ref, ln2_b_ref,
    w1_ref, b1_ref, w2_ref, b2_ref,
    o_ref,
    h1_scr,
    *, num_heads: int, head_size: int, block_batch: int, seq_len: int,
    ffn_chunks: int,
):
    D = num_heads * head_size
    bb, T = block_batch, seq_len
    M = bb * T
    Mc = M // ffn_chunks


    def _ffn_chunk(c):
        rows = pl.ds(c * Mc, Mc)
        h1c = h1_scr[rows, :]
        hnc = _layer_norm_f32(h1c, ln2_g_ref[...], ln2_b_ref[...])
        fc = jnp.dot(hnc.astype(jnp.bfloat16), w1_ref[...],
                     preferred_element_type=jnp.float32) + b1_ref[...]
        fc = jnp.maximum(fc, 0.0)
        oc = jnp.dot(fc.astype(jnp.bfloat16), w2_ref[...],
                     preferred_element_type=jnp.float32) + b2_ref[...]
        o_ref[rows, :] = (oc + h1c).astype(o_ref.dtype)

    _ffn_chunk(0)

    x = x_ref[...].astype(jnp.float32)
    xn = _layer_norm_f32(x, ln1_g_ref[...], ln1_b_ref[...])
    qkv = jnp.dot(xn.astype(jnp.bfloat16), wqkv_ref[...],
                  preferred_element_type=jnp.float32) + bqkv_ref[...]

    kbias = kbias_ref[...]
    _chunk_at = {1: 1, 2: 2, 4: 3, 5: 4, 7: 5, 8: 6, 10: 7} if ffn_chunks == 8 \
        else {3: 1, 6: 2, 9: 3}
    ctxs = []
    for h in range(num_heads):
        if h in _chunk_at:
            _ffn_chunk(_chunk_at[h])
        lo = h * head_size
        qh = qkv[:, lo:lo + head_size].astype(jnp.bfloat16).reshape(bb, T, head_size)
        kh = qkv[:, D + lo:D + lo + head_size].astype(jnp.bfloat16).reshape(bb, T, head_size)
        vh = qkv[:, 2 * D + lo:2 * D + lo + head_size].astype(jnp.bfloat16).reshape(bb, T, head_size)
        s = jax.lax.dot_general(qh, kh, (((2,), (2,)), ((0,), (0,))),
                                preferred_element_type=jnp.float32)
        s = s + kbias
        p = jnp.exp(jnp.minimum(s, 80.0))
        r = pl.reciprocal(jnp.sum(p, axis=-1, keepdims=True), approx=True)
        ctx = jax.lax.dot_general(p.astype(jnp.bfloat16), vh,
                                  (((2,), (1,)), ((0,), (0,))),
                                  preferred_element_type=jnp.float32)
        ctx = ctx * r
        ctxs.append(ctx.reshape(M, head_size).astype(jnp.bfloat16))
    ctx_full = jnp.concatenate(ctxs, axis=-1)

    h1_scr[...] = x + bo_ref[...] + jnp.dot(ctx_full, wo_ref[...],
                                            preferred_element_type=jnp.float32)


def kernel(x, mask, ln1_g, ln1_b, wq, bq, wk, bk, wv, bv, wo, bo,
           ln2_g, ln2_b, w1, b1, w2, b2):
    num_heads = 12
    B, T, D = x.shape
    head_size = D // num_heads
    F = w1.shape[1]

    block_batch = max(1, min(B, 1024 // max(T, 1)))
    while block_batch > 1 and (B % block_batch != 0 or (block_batch * T) % 8 != 0):
        block_batch -= 1
    M = block_batch * T
    num_blocks = B // block_batch
    num_cores = 2 if num_blocks % 2 == 0 else 1
    steps_per_core = num_blocks // num_cores
    ffn_chunks = 4 if M % (4 * 8) == 0 else 1

    scale = 1.0 / math.sqrt(head_size)
    wqkv = jnp.concatenate([wq * scale, wk, wv], axis=1).astype(jnp.bfloat16)
    bqkv = jnp.concatenate([bq * scale, bk, bv], axis=1)
    wo_b = wo.astype(jnp.bfloat16)
    w1_b = w1.astype(jnp.bfloat16)
    w2_b = w2.astype(jnp.bfloat16)

    mask_full = jnp.concatenate([mask[:, :, 0:1], mask], axis=-1)
    kbias = jnp.where(mask_full, 0.0, _NEG_INF).astype(jnp.float32)

    x2 = x.reshape(B * T, D)
    spc = steps_per_core
    cur = lambda c, j: (c * spc + jnp.minimum(j, spc - 1), 0)
    prev = lambda c, j: (c * spc + jnp.maximum(j - 1, 0), 0)
    cur3 = lambda c, j: (c * spc + jnp.minimum(j, spc - 1), 0, 0)
    const = lambda c, j: (0, 0)

    _kernel_fn = functools.partial(
        _encoder_layer_kernel, num_heads=num_heads, head_size=head_size,
        block_batch=block_batch, seq_len=T, ffn_chunks=ffn_chunks)

    out = pl.pallas_call(
        _kernel_fn,
        out_shape=jax.ShapeDtypeStruct((B * T, D), x.dtype),
        grid=(num_cores, spc + 1),
        in_specs=[
            pl.BlockSpec((M, D), cur),
            pl.BlockSpec((block_batch, 1, T), cur3),
            pl.BlockSpec((1, D), const),
            pl.BlockSpec((1, D), const),
            pl.BlockSpec((D, 3 * D), const),
            pl.BlockSpec((1, 3 * D), const),
            pl.BlockSpec((D, D), const),
            pl.BlockSpec((1, D), const),
            pl.BlockSpec((1, D), const),
            pl.BlockSpec((1, D), const),
            pl.BlockSpec((D, F), const),
            pl.BlockSpec((1, F), const),
            pl.BlockSpec((F, D), const),
            pl.BlockSpec((1, D), const),
        ],
        out_specs=pl.BlockSpec((M, D), prev),
        scratch_shapes=[pltpu.VMEM((M, D), jnp.float32)],
        compiler_params=pltpu.CompilerParams(
            dimension_semantics=("parallel", "arbitrary"),
            vmem_limit_bytes=60 * 1024 * 1024),
    )(x2, kbias,
      ln1_g, ln1_b,
      wqkv, bqkv, wo_b, bo,
      ln2_g, ln2_b,
      w1_b, b1, w2_b, b2)
    return out.reshape(B, T, D)
